# Initial kernel scaffold; baseline (speedup 1.0000x reference)
import jax
import jax.numpy as jnp
from jax.experimental import pallas as pl


def kernel(x, conv1_w, conv1_shift, conv2_w, conv2_shift, conv3_w, conv3_shift, conv4_w, conv4_shift, fc1_w, fc1_b, fc2_w, fc2_b, fc3_w, fc3_b):
    raise NotImplementedError("write your pallas kernel here")


# speedup vs baseline: 1.7466x; 1.7466x over previous
import jax
import jax.numpy as jnp
from jax.experimental import pallas as pl
from jax.experimental.pallas import tpu as pltpu

B = 8

_LAYERS = [
    (154, 1, 4),
    (78, 4, 16),
    (38, 16, 32),
    (18, 32, 64),
]


def _bands(w2d, W, Cin, Cout):
    Wout = W - 2
    w = w2d[:Cout].astype(jnp.float32).reshape(Cout, 3, 3, Cin)
    outs = []
    for di in range(3):
        band = jnp.zeros((Cin * W, Cout * Wout), jnp.float32)
        for dj in range(3):
            eye = jnp.eye(W, Wout, k=-dj, dtype=jnp.float32)
            band = band + jnp.einsum(
                "oi,uc->iuoc", w[:, di, dj, :], eye).reshape(Cin * W, Cout * Wout)
        outs.append(band.astype(jnp.bfloat16))
    return outs


def _sel_cols(C, Wout, Wn, off):
    wp = Wout // 2
    q = jnp.arange(C * wp)
    rows = (q // wp) * Wout + 2 * (q % wp)
    cols = (q // wp) * Wn + off + (q % wp)
    sel = jnp.zeros((C * Wout, C * Wn), jnp.float32)
    return sel.at[rows, cols].set(1.0)


def _sel_rows(H, Hn, off):
    hp = (H - 2) // 2
    q = jnp.arange(B * hp)
    rows = (q // hp) * Hn + off + (q % hp)
    cols = (q // hp) * H + 2 * (q % hp)
    sel = jnp.zeros((B * Hn, B * H), jnp.float32)
    return sel.at[rows, cols].set(1.0)


def _net_kernel(x_ref,
                b10, b11, b12, b20, b21, b22, b30, b31, b32, b40, b41, b42,
                s1, s2, s3, s4,
                r1, c1, r2, c2, r3, c3, r4, c4,
                w1, bb1, w2, bb2, w3, bb3,
                o_ref,
                a1b, y1, a2b, y2, a3b, y3, a4b, y4, fcin):
    f32 = jnp.float32

    def conv(src_b, bands, s_ref, y_scr, M):
        acc = jnp.dot(src_b[0:M, :], bands[0][...], preferred_element_type=f32)
        acc = acc + jnp.dot(src_b[1:M + 1, :], bands[1][...],
                            preferred_element_type=f32)
        acc = acc + jnp.dot(src_b[2:M + 2, :], bands[2][...],
                            preferred_element_type=f32)
        y_scr[0:M, :] = jnp.maximum(acc + s_ref[...], 0.0)

    def pool(y_scr, r_ref, c_ref, BH, CW):
        yv = y_scr[...]
        m1 = jnp.maximum(yv, pltpu.roll(yv, BH - 1, axis=0))
        m = jnp.maximum(m1, pltpu.roll(m1, CW - 1, axis=1))
        t = jnp.dot(m, c_ref[...], preferred_element_type=f32)
        return jnp.dot(r_ref[...], t, preferred_element_type=f32)

    a1b[...] = x_ref[...].astype(jnp.bfloat16)
    conv(a1b, (b10, b11, b12), s1, y1, B * 154 - 2)
    a2b[...] = pool(y1, r1, c1, B * 154, 608).astype(jnp.bfloat16)

    conv(a2b, (b20, b21, b22), s2, y2, B * 78 - 2)
    a3b[...] = pool(y2, r2, c2, B * 78, 1216).astype(jnp.bfloat16)

    conv(a3b, (b30, b31, b32), s3, y3, B * 38 - 2)
    a4b[...] = pool(y3, r3, c3, B * 38, 1152).astype(jnp.bfloat16)

    conv(a4b, (b40, b41, b42), s4, y4, B * 18 - 2)
    h4 = pool(y4, r4, c4, B * 18, 1024)

    for b in range(B):
        for i in range(8):
            fcin[b:b + 1, i * 512:(i + 1) * 512] = \
                h4[b * 9 + i:b * 9 + i + 1, :].astype(jnp.bfloat16)

    z1 = jnp.dot(fcin[...], w1[...], preferred_element_type=f32) + bb1[...]
    z2 = jnp.dot(z1.astype(jnp.bfloat16), w2[...],
                 preferred_element_type=f32) + bb2[...]
    z3 = jnp.dot(z2.astype(jnp.bfloat16), w3[...],
                 preferred_element_type=f32) + bb3[...]
    o_ref[...] = z3


def kernel(x, conv1_w, conv1_shift, conv2_w, conv2_shift, conv3_w, conv3_shift,
           conv4_w, conv4_shift, fc1_w, fc1_b, fc2_w, fc2_b, fc3_w, fc3_b):
    N = x.shape[0]
    xr = x.astype(jnp.float32).reshape(N * 154, 154)

    bands = []
    for (W, Cin, Cout), w2d in zip(
            _LAYERS, (conv1_w, conv2_w, conv3_w, conv4_w)):
        bands.extend(_bands(w2d, W, Cin, Cout))

    shifts = []
    for (W, Cin, Cout), sh in zip(
            _LAYERS, (conv1_shift, conv2_shift, conv3_shift, conv4_shift)):
        shifts.append(jnp.repeat(sh[:Cout, 0], W - 2)[None, :])

    sels = [
        _sel_rows(154, 78, 1), _sel_cols(4, 152, 78, 1),
        _sel_rows(78, 38, 0), _sel_cols(16, 76, 38, 0),
        _sel_rows(38, 18, 0), _sel_cols(32, 36, 18, 0),
        _sel_rows(18, 9, 0), _sel_cols(64, 16, 8, 0),
    ]

    n = jnp.arange(64 * 8 * 8)
    g = (n % 512) // 8 * 64 + (n // 512) * 8 + (n % 8)
    w1p = fc1_w[g, :]

    grid = (N // B,)
    row_spec = lambda r: pl.BlockSpec(r.shape, lambda i: (0, 0))

    out = pl.pallas_call(
        _net_kernel,
        out_shape=jax.ShapeDtypeStruct((N, 10), jnp.float32),
        grid_spec=pltpu.PrefetchScalarGridSpec(
            num_scalar_prefetch=0,
            grid=grid,
            in_specs=[pl.BlockSpec((B * 154, 154), lambda i: (i, 0))]
            + [row_spec(b) for b in bands]
            + [row_spec(s) for s in shifts]
            + [row_spec(s) for s in sels]
            + [row_spec(w1p), row_spec(fc1_b), row_spec(fc2_w),
               row_spec(fc2_b), row_spec(fc3_w), row_spec(fc3_b)],
            out_specs=pl.BlockSpec((B, 10), lambda i: (i, 0)),
            scratch_shapes=[
                pltpu.VMEM((B * 154, 154), jnp.bfloat16),
                pltpu.VMEM((B * 154, 608), jnp.float32),
                pltpu.VMEM((B * 78, 312), jnp.bfloat16),
                pltpu.VMEM((B * 78, 1216), jnp.float32),
                pltpu.VMEM((B * 38, 608), jnp.bfloat16),
                pltpu.VMEM((B * 38, 1152), jnp.float32),
                pltpu.VMEM((B * 18, 576), jnp.bfloat16),
                pltpu.VMEM((B * 18, 1024), jnp.float32),
                pltpu.VMEM((B, 4096), jnp.bfloat16),
            ],
        ),
        compiler_params=pltpu.CompilerParams(
            dimension_semantics=("parallel",)),
    )(xr, *bands, *shifts, *sels, w1p, fc1_b, fc2_w, fc2_b, fc3_w, fc3_b)
    return out
